# baseline (speedup 1.0000x reference)
import functools

import jax
import jax.numpy as jnp
from jax import lax
from jax.experimental import pallas as pl
from jax.experimental.pallas import tpu as pltpu
from jax.experimental.pallas import tpu_sc as plsc

C = 3
T = 300
S = 32
ROW = 256 * 256
CHUNK = ROW // 2
NC, NS_SUB = 2, 16
NW = NC * NS_SUB
FRAMES = C * S
FPW = FRAMES // NW
CPW = FPW * (ROW // CHUNK)
NBUF = 3


def _sc_subsample(x_flat):
    mesh = plsc.VectorSubcoreMesh(core_axis_name="c", subcore_axis_name="s")

    @functools.partial(
        pl.kernel,
        mesh=mesh,
        out_type=jax.ShapeDtypeStruct((FRAMES * ROW,), jnp.float32),
        scratch_types=[
            pltpu.VMEM((CHUNK,), jnp.float32),
            pltpu.VMEM((CHUNK,), jnp.float32),
            pltpu.VMEM((CHUNK,), jnp.float32),
            pltpu.SemaphoreType.DMA((NBUF,)),
            pltpu.SemaphoreType.DMA((NBUF,)),
        ],
    )
    def k(x_hbm, out_hbm, buf0, buf1, buf2, rsem, wsem):
        bufs = (buf0, buf1, buf2)
        wid = lax.axis_index("s") * NC + lax.axis_index("c")
        f0 = wid * FPW

        def src_word(kk):
            f = f0 + (kk // 2)
            c = f // S
            i = f - c * S
            src = (i * (T - 1)) // (S - 1)
            return (c * T + src) * ROW + (kk % 2) * CHUNK

        def dst_word(kk):
            f = f0 + (kk // 2)
            return f * ROW + (kk % 2) * CHUNK

        def read(kk):
            return pltpu.async_copy(
                x_hbm.at[pl.ds(src_word(kk), CHUNK)],
                bufs[kk % NBUF], rsem.at[kk % NBUF])

        def write(kk):
            return pltpu.async_copy(
                bufs[kk % NBUF],
                out_hbm.at[pl.ds(dst_word(kk), CHUNK)], wsem.at[kk % NBUF])

        reads = [None] * CPW
        writes = [None] * CPW
        for kk in range(min(NBUF, CPW)):
            reads[kk] = read(kk)
        for kk in range(CPW):
            reads[kk].wait()
            writes[kk] = write(kk)
            nxt = kk + 2
            if kk >= 1 and nxt < CPW:
                writes[kk - 1].wait()
                reads[nxt] = read(nxt)
        for kk in range(max(0, CPW - NBUF), CPW):
            if writes[kk] is not None:
                writes[kk].wait()

    return k(x_flat)


def kernel(x):
    x_flat = x.reshape(C * T * ROW)
    out = _sc_subsample(x_flat)
    return out.reshape(C, S, 256, 256)
